# baseline (speedup 1.0000x reference)
import functools

import jax
import jax.numpy as jnp
import numpy as np
from jax import lax
from jax.experimental import pallas as pl
from jax.experimental.pallas import tpu as pltpu
from jax.experimental.pallas import tpu_sc as plsc

VOCAB = 100000
D = 768
L_SEQ = 2048
B = 4

NC = 2
NS = 16
NW = NC * NS
POS_PER_W = L_SEQ // NW
CH = 32
CPB = POS_PER_W // CH
NCHUNK = B * CPB
NBUF = 3
LANES = 16
KSTEPS = D // LANES


def _pos_encoding():
    even_i = np.arange(0, D, 2, dtype=np.float32)
    denominator = np.power(np.float32(10000.0), even_i / np.float32(D))
    position = np.arange(L_SEQ, dtype=np.float32).reshape(L_SEQ, 1)
    even_pe = np.sin(position / denominator, dtype=np.float32)
    odd_pe = np.cos(position / denominator, dtype=np.float32)
    stacked = np.stack([even_pe, odd_pe], axis=2)
    return stacked.reshape(L_SEQ, D).astype(np.float32)


_PE = _pos_encoding().reshape(-1)


def _sc_body(tok_hbm, pe_hbm, table_hbm, out_hbm,
             idx_v, pe_v, row_v, psem, gsem, wsem):
    w = lax.axis_index("s") * NC + lax.axis_index("c")
    pos_base = w * POS_PER_W

    pe_desc = pltpu.async_copy(
        pe_hbm.at[pl.ds(pos_base * D, POS_PER_W * D)], pe_v, psem)
    for b in range(B):
        pltpu.sync_copy(tok_hbm.at[b, pl.ds(pos_base, POS_PER_W)],
                        idx_v.at[b])

    def gather(c):
        b, q = c // CPB, c % CPB
        return pltpu.async_copy(
            table_hbm.at[idx_v.at[b, pl.ds(q * CH, CH)]],
            row_v.at[c % NBUF], gsem.at[c % NBUF])

    gd = {0: gather(0), 1: gather(1)}
    wd = {}
    pe_desc.wait()

    for c in range(NCHUNK):
        b, q = c // CPB, c % CPB
        j = c % NBUF
        gd[c].wait()
        if c >= 1:
            wd[c - 1].wait()
        if c + 2 < NCHUNK:
            gd[c + 2] = gather(c + 2)

        def add_row(r, _, j=j, q=q):
            pe_off = (q * CH + r) * D
            for k in range(KSTEPS):
                plsc.addupdate(row_v.at[j, r, pl.ds(k * LANES, LANES)],
                               pe_v[pl.ds(pe_off + k * LANES, LANES)])
            return _

        lax.fori_loop(0, CH, add_row, 0)

        wd[c] = pltpu.async_copy(
            row_v.at[j],
            out_hbm.at[b, pl.ds(pos_base + q * CH, CH)],
            wsem.at[j])

    wd[NCHUNK - 1].wait()


@jax.jit
def _sc_embed(tokens, pe, table):
    mesh = plsc.VectorSubcoreMesh(core_axis_name="c", subcore_axis_name="s")
    k = pl.kernel(
        _sc_body,
        out_type=jax.ShapeDtypeStruct((B, L_SEQ, D), jnp.float32),
        mesh=mesh,
        scratch_types=[
            pltpu.VMEM((B, POS_PER_W), jnp.int32),
            pltpu.VMEM((POS_PER_W * D,), jnp.float32),
            pltpu.VMEM((NBUF, CH, D), jnp.float32),
            pltpu.SemaphoreType.DMA,
            pltpu.SemaphoreType.DMA((NBUF,)),
            pltpu.SemaphoreType.DMA((NBUF,)),
        ],
    )
    return k(tokens, pe, table)


def kernel(tokens, table):
    return _sc_embed(tokens, _PE, table)
